# baseline (speedup 1.0000x reference)
import functools

import jax
import jax.numpy as jnp
from jax import lax
from jax.experimental import pallas as pl
from jax.experimental.pallas import tpu as pltpu
from jax.experimental.pallas import tpu_sc as plsc

NUM_EMBEDDINGS = 1000000
EMBEDDING_DIM = 64
CHUNK = 512


@functools.cache
def _build(batch: int):
    info = plsc.get_sparse_core_info()
    nc, ns = info.num_cores, info.num_subcores
    nw = nc * ns
    assert batch % (nw * CHUNK) == 0
    per_worker = batch // nw
    n_chunks = per_worker // CHUNK

    mesh = plsc.VectorSubcoreMesh(core_axis_name="c", subcore_axis_name="s")

    @functools.partial(
        pl.kernel,
        mesh=mesh,
        out_type=jax.ShapeDtypeStruct((batch, EMBEDDING_DIM), jnp.float32),
        scratch_types=[
            pltpu.VMEM((CHUNK,), jnp.int32),
            pltpu.VMEM((CHUNK, EMBEDDING_DIM), jnp.float32),
            pltpu.SemaphoreType.DMA,
        ],
        compiler_params=pltpu.CompilerParams(use_tc_tiling_on_sc=False),
    )
    def gather_kernel(idx_hbm, table_hbm, out_hbm, idx_v, rows_v, sem):
        wid = lax.axis_index("s") * nc + lax.axis_index("c")
        base = wid * per_worker

        def body(i, carry):
            off = base + i * CHUNK
            pltpu.sync_copy(idx_hbm.at[pl.ds(off, CHUNK)], idx_v)
            pltpu.async_copy(table_hbm.at[idx_v], rows_v, sem).wait()
            pltpu.sync_copy(rows_v, out_hbm.at[pl.ds(off, CHUNK)])
            return carry

        lax.fori_loop(0, n_chunks, body, 0)

    return gather_kernel


def kernel(words, weight):
    b, s = words.shape
    flat_idx = words.reshape(b * s).astype(jnp.int32)
    out = _build(b * s)(flat_idx, weight)
    return out.reshape(b, s, EMBEDDING_DIM)


# speedup vs baseline: 1.0479x; 1.0479x over previous
import functools

import jax
import jax.numpy as jnp
from jax import lax
from jax.experimental import pallas as pl
from jax.experimental.pallas import tpu as pltpu
from jax.experimental.pallas import tpu_sc as plsc

NUM_EMBEDDINGS = 1000000
EMBEDDING_DIM = 64
CHUNK = 320
NBUF = 4


@functools.cache
def _build(batch: int):
    info = plsc.get_sparse_core_info()
    nc, ns = info.num_cores, info.num_subcores
    nw = nc * ns
    assert batch % (nw * CHUNK) == 0
    per_worker = batch // nw
    n_chunks = per_worker // CHUNK
    assert n_chunks % NBUF == 0 and n_chunks >= 2 * NBUF

    mesh = plsc.VectorSubcoreMesh(core_axis_name="c", subcore_axis_name="s")

    @functools.partial(
        pl.kernel,
        mesh=mesh,
        out_type=jax.ShapeDtypeStruct((batch, EMBEDDING_DIM), jnp.float32),
        scratch_types=[
            pltpu.VMEM((n_chunks, CHUNK), jnp.int32),
            [pltpu.VMEM((CHUNK, EMBEDDING_DIM), jnp.float32)] * NBUF,
            [pltpu.SemaphoreType.DMA] * NBUF,
            [pltpu.SemaphoreType.DMA] * NBUF,
        ],
        compiler_params=pltpu.CompilerParams(use_tc_tiling_on_sc=False),
    )
    def gather_kernel(idx_hbm, table_hbm, out_hbm, idx_v, rows, gsem, ssem):
        wid = lax.axis_index("s") * nc + lax.axis_index("c")
        base = wid * per_worker

        pltpu.sync_copy(idx_hbm.at[pl.ds(wid * n_chunks, n_chunks)], idx_v)

        def start_gather(b, t):
            pltpu.async_copy(table_hbm.at[idx_v.at[t]], rows[b], gsem[b])

        def wait_gather(b, t):
            pltpu.make_async_copy(table_hbm.at[idx_v.at[t]], rows[b],
                                  gsem[b]).wait()

        def out_slice(t):
            return out_hbm.at[pl.ds(base + t * CHUNK, CHUNK)]

        def start_store(b, t):
            pltpu.async_copy(rows[b], out_slice(t), ssem[b])

        def wait_store(b, t):
            pltpu.make_async_copy(rows[b], out_slice(t), ssem[b]).wait()

        for t in range(NBUF):
            start_gather(t, t)
            if t >= 2:
                wait_gather(t - 2, t - 2)
                start_store(t - 2, t - 2)

        @pl.loop(NBUF, n_chunks, step=NBUF)
        def _(q):
            for b in range(NBUF):
                t = q + b
                wait_store(b, t - NBUF)
                start_gather(b, t)
                b2 = (b - 2) % NBUF
                wait_gather(b2, t - 2)
                start_store(b2, t - 2)

        for t in range(n_chunks - 2, n_chunks):
            b = t % NBUF
            wait_gather(b, t)
            start_store(b, t)
        for t in range(n_chunks - NBUF, n_chunks):
            wait_store(t % NBUF, t)

    return gather_kernel


def kernel(words, weight):
    b, s = words.shape
    flat_idx = words.reshape(-1, CHUNK).astype(jnp.int32)
    out = _build(b * s)(flat_idx, weight)
    return out.reshape(b, s, EMBEDDING_DIM)
